# Initial kernel scaffold; baseline (speedup 1.0000x reference)
import jax
import jax.numpy as jnp
from jax.experimental import pallas as pl


def kernel(x, packed_params):
    raise NotImplementedError("write your pallas kernel here")


# speedup vs baseline: 1.2390x; 1.2390x over previous
import functools

import jax
import jax.numpy as jnp
from jax.experimental import pallas as pl
from jax.experimental.pallas import tpu as pltpu

IN_DIM = 10
HID_DIM = 10
OUT_DIM = 5
LANES = 128

_W1_OFF = 0
_B1_OFF = 16
_W2_OFF = 24
_B2_OFF = 152

_G = 8
_INP = _G * IN_DIM
_HIDP = _G * HID_DIM
_OUTP = _G * OUT_DIM
_W1K_OFF = 0
_B1K_OFF = _INP
_W2K_OFF = 88
_B2K_OFF = _W2K_OFF + LANES
_PROWS = 224


def _repack(packed_params):
    pp = packed_params.astype(jnp.float32)
    w1 = pp[_W1_OFF:_W1_OFF + IN_DIM, :HID_DIM]
    b1 = pp[_B1_OFF, :HID_DIM]
    w2 = pp[_W2_OFF:_W2_OFF + HID_DIM, :OUT_DIM]
    b2 = pp[_B2_OFF, :OUT_DIM]

    eye = jnp.eye(_G, dtype=jnp.float32)
    w1k = jnp.kron(eye, w1)
    w2k = jnp.kron(eye, w2)
    b1k = jnp.tile(b1, _G)
    b2k = jnp.tile(b2, _G)

    p = jnp.zeros((_PROWS, LANES), jnp.float32)
    p = p.at[_W1K_OFF:_W1K_OFF + _INP, :_HIDP].set(w1k)
    p = p.at[_B1K_OFF, :_HIDP].set(b1k)
    p = p.at[_W2K_OFF:_W2K_OFF + _HIDP, :_OUTP].set(w2k)
    p = p.at[_B2K_OFF, :_OUTP].set(b2k)
    return p


def _mlp_kernel(x_ref, p_ref, o_ref):
    x = x_ref[...]
    w1 = p_ref[_W1K_OFF:_W1K_OFF + _INP, :]
    b1 = p_ref[_B1K_OFF:_B1K_OFF + 1, :]
    w2 = p_ref[_W2K_OFF:_W2K_OFF + LANES, :]
    b2 = p_ref[_B2K_OFF:_B2K_OFF + 1, :]

    h = jnp.dot(x, w1, preferred_element_type=jnp.float32) + b1
    h = jnp.maximum(h, 0.0)
    y = jnp.dot(h, w2, preferred_element_type=jnp.float32) + b2
    o_ref[...] = y[:, :_OUTP].astype(o_ref.dtype)


def _mlp_kernel_unpacked(x_ref, p_ref, o_ref):
    x = x_ref[...]
    w1 = p_ref[_W1_OFF:_W1_OFF + IN_DIM, :]
    b1 = p_ref[_B1_OFF:_B1_OFF + 1, :]
    w2 = p_ref[_W2_OFF:_W2_OFF + LANES, :]
    b2 = p_ref[_B2_OFF:_B2_OFF + 1, :]
    h = jnp.maximum(jnp.dot(x, w1, preferred_element_type=jnp.float32) + b1, 0.0)
    y = jnp.dot(h, w2, preferred_element_type=jnp.float32) + b2
    o_ref[...] = y[:, :OUT_DIM].astype(o_ref.dtype)


@functools.partial(jax.jit, static_argnames=("tile_r",))
def _forward(x, packed_params, *, tile_r=4096):
    B = x.shape[0]

    packed = B % _G == 0
    if packed:
        xr = x.reshape(B // _G, _INP)
        p = _repack(packed_params)
        inp, outp = _INP, _OUTP
        prows = _PROWS
        body = _mlp_kernel
    else:
        xr = x
        p = packed_params
        inp, outp = IN_DIM, OUT_DIM
        prows = packed_params.shape[0]
        body = _mlp_kernel_unpacked
    Br = xr.shape[0]

    tile = min(tile_r, ((Br + 7) // 8) * 8)
    grid = (pl.cdiv(Br, tile),)

    cost = pl.CostEstimate(
        flops=2 * Br * (inp * LANES + LANES * LANES),
        transcendentals=0,
        bytes_accessed=Br * (inp + outp) * 4 + prows * LANES * 4,
    )

    out = pl.pallas_call(
        body,
        out_shape=jax.ShapeDtypeStruct((Br, outp), jnp.float32),
        grid=grid,
        in_specs=[
            pl.BlockSpec((tile, inp), lambda i: (i, 0)),
            pl.BlockSpec((prows, LANES), lambda i: (0, 0)),
        ],
        out_specs=pl.BlockSpec((tile, outp), lambda i: (i, 0)),
        compiler_params=pltpu.CompilerParams(
            dimension_semantics=("parallel",),
        ),
        cost_estimate=cost,
    )(xr, p)

    return out.reshape(B, OUT_DIM)


def kernel(x, packed_params):
    return _forward(x, packed_params)


# speedup vs baseline: 24.5690x; 19.8295x over previous
import functools

import jax
import jax.numpy as jnp
from jax import lax
from jax.experimental import pallas as pl
from jax.experimental.pallas import tpu as pltpu

IN_DIM = 10
HID_DIM = 10
OUT_DIM = 5
LANES = 128

_W1_OFF = 0
_B1_OFF = 16
_W2_OFF = 24
_B2_OFF = 152

_CONTRACT0 = (((0,), (0,)), ((), ()))


def _mlp_t_kernel(x_ref, p_ref, o_ref):
    x = x_ref[...]
    w1 = p_ref[_W1_OFF:_W1_OFF + IN_DIM, :HID_DIM]
    b1 = p_ref[_B1_OFF:_B1_OFF + 1, :HID_DIM]
    w2 = p_ref[_W2_OFF:_W2_OFF + HID_DIM, :OUT_DIM]
    b2 = p_ref[_B2_OFF:_B2_OFF + 1, :OUT_DIM]

    ones = jnp.ones((1, x.shape[1]), jnp.float32)
    f32 = jnp.float32
    h = (lax.dot_general(w1, x, _CONTRACT0, preferred_element_type=f32)
         + lax.dot_general(b1, ones, _CONTRACT0, preferred_element_type=f32))
    h = jnp.maximum(h, 0.0)
    y = (lax.dot_general(w2, h, _CONTRACT0, preferred_element_type=f32)
         + lax.dot_general(b2, ones, _CONTRACT0, preferred_element_type=f32))
    o_ref[...] = y.astype(o_ref.dtype)


@functools.partial(jax.jit, static_argnames=("tile_b",))
def _forward(x, packed_params, *, tile_b=32768):
    B = x.shape[0]
    xt = x.T
    prows = packed_params.shape[0]

    tile = min(tile_b, ((B + LANES - 1) // LANES) * LANES)
    grid = (pl.cdiv(B, tile),)

    cost = pl.CostEstimate(
        flops=2 * B * (IN_DIM + 1) * (HID_DIM + OUT_DIM),
        transcendentals=0,
        bytes_accessed=B * (IN_DIM + OUT_DIM) * 4 + prows * LANES * 4,
    )

    yt = pl.pallas_call(
        _mlp_t_kernel,
        out_shape=jax.ShapeDtypeStruct((OUT_DIM, B), jnp.float32),
        grid=grid,
        in_specs=[
            pl.BlockSpec((IN_DIM, tile), lambda i: (0, i)),
            pl.BlockSpec((prows, LANES), lambda i: (0, 0)),
        ],
        out_specs=pl.BlockSpec((OUT_DIM, tile), lambda i: (0, i)),
        compiler_params=pltpu.CompilerParams(
            dimension_semantics=("parallel",),
        ),
        cost_estimate=cost,
    )(xt, packed_params)

    return yt.T


def kernel(x, packed_params):
    return _forward(x, packed_params)


# speedup vs baseline: 30.6630x; 1.2480x over previous
import functools

import jax
import jax.numpy as jnp
from jax import lax
from jax.experimental import pallas as pl
from jax.experimental.pallas import tpu as pltpu

IN_DIM = 10
HID_DIM = 10
OUT_DIM = 5
LANES = 128

_W1_OFF = 0
_B1_OFF = 16
_W2_OFF = 24
_B2_OFF = 152

_CONTRACT0 = (((0,), (0,)), ((), ()))


def _mlp_t_kernel(x_ref, p_ref, o_ref):
    x = x_ref[...]
    w1 = p_ref[_W1_OFF:_W1_OFF + IN_DIM, :HID_DIM]
    b1 = p_ref[_B1_OFF:_B1_OFF + 1, :HID_DIM]
    w2 = p_ref[_W2_OFF:_W2_OFF + HID_DIM, :OUT_DIM]
    b2 = p_ref[_B2_OFF:_B2_OFF + 1, :OUT_DIM]

    ones = jnp.ones((1, x.shape[1]), jnp.float32)
    f32 = jnp.float32
    h = (lax.dot_general(w1, x, _CONTRACT0, preferred_element_type=f32)
         + lax.dot_general(b1, ones, _CONTRACT0, preferred_element_type=f32))
    h = jnp.maximum(h, 0.0)
    y = (lax.dot_general(w2, h, _CONTRACT0, preferred_element_type=f32)
         + lax.dot_general(b2, ones, _CONTRACT0, preferred_element_type=f32))
    o_ref[...] = y.astype(o_ref.dtype)


@functools.partial(jax.jit, static_argnames=("tile_b",))
def _forward(x, packed_params, *, tile_b=65536):
    B = x.shape[0]
    xt = x.T
    prows = packed_params.shape[0]

    tile = min(tile_b, ((B + LANES - 1) // LANES) * LANES)
    grid = (pl.cdiv(B, tile),)

    cost = pl.CostEstimate(
        flops=2 * B * (IN_DIM + 1) * (HID_DIM + OUT_DIM),
        transcendentals=0,
        bytes_accessed=B * (IN_DIM + OUT_DIM) * 4 + prows * LANES * 4,
    )

    yt = pl.pallas_call(
        _mlp_t_kernel,
        out_shape=jax.ShapeDtypeStruct((OUT_DIM, B), jnp.float32),
        grid=grid,
        in_specs=[
            pl.BlockSpec((IN_DIM, tile), lambda i: (0, i)),
            pl.BlockSpec((prows, LANES), lambda i: (0, 0)),
        ],
        out_specs=pl.BlockSpec((OUT_DIM, tile), lambda i: (0, i)),
        compiler_params=pltpu.CompilerParams(
            dimension_semantics=("parallel",),
        ),
        cost_estimate=cost,
    )(xt, packed_params)

    return yt.T


def kernel(x, packed_params):
    return _forward(x, packed_params)


# speedup vs baseline: 33.0055x; 1.0764x over previous
import functools

import jax
import jax.numpy as jnp
from jax import lax
from jax.experimental import pallas as pl
from jax.experimental.pallas import tpu as pltpu

IN_DIM = 10
HID_DIM = 10
OUT_DIM = 5
LANES = 128

_W1_OFF = 0
_B1_OFF = 16
_W2_OFF = 24
_B2_OFF = 152

_CONTRACT0 = (((0,), (0,)), ((), ()))


def _mlp_t_kernel(x_ref, p_ref, o_ref):
    x = x_ref[...]
    w1 = p_ref[_W1_OFF:_W1_OFF + IN_DIM, :HID_DIM]
    b1 = p_ref[_B1_OFF:_B1_OFF + 1, :HID_DIM]
    w2 = p_ref[_W2_OFF:_W2_OFF + HID_DIM, :OUT_DIM]
    b2 = p_ref[_B2_OFF:_B2_OFF + 1, :OUT_DIM]

    ones = jnp.ones((1, x.shape[1]), jnp.float32)
    f32 = jnp.float32
    h = (lax.dot_general(w1, x, _CONTRACT0, preferred_element_type=f32)
         + lax.dot_general(b1, ones, _CONTRACT0, preferred_element_type=f32))
    h = jnp.maximum(h, 0.0)
    y = (lax.dot_general(w2, h, _CONTRACT0, preferred_element_type=f32)
         + lax.dot_general(b2, ones, _CONTRACT0, preferred_element_type=f32))
    o_ref[...] = y.astype(o_ref.dtype)


@functools.partial(jax.jit, static_argnames=("tile_b",))
def _forward(x, packed_params, *, tile_b=131072):
    B = x.shape[0]
    xt = x.T
    prows = packed_params.shape[0]

    tile = min(tile_b, ((B + LANES - 1) // LANES) * LANES)
    grid = (pl.cdiv(B, tile),)

    cost = pl.CostEstimate(
        flops=2 * B * (IN_DIM + 1) * (HID_DIM + OUT_DIM),
        transcendentals=0,
        bytes_accessed=B * (IN_DIM + OUT_DIM) * 4 + prows * LANES * 4,
    )

    yt = pl.pallas_call(
        _mlp_t_kernel,
        out_shape=jax.ShapeDtypeStruct((OUT_DIM, B), jnp.float32),
        grid=grid,
        in_specs=[
            pl.BlockSpec((IN_DIM, tile), lambda i: (0, i)),
            pl.BlockSpec((prows, LANES), lambda i: (0, 0)),
        ],
        out_specs=pl.BlockSpec((OUT_DIM, tile), lambda i: (0, i)),
        compiler_params=pltpu.CompilerParams(
            dimension_semantics=("parallel",),
        ),
        cost_estimate=cost,
    )(xt, packed_params)

    return yt.T


def kernel(x, packed_params):
    return _forward(x, packed_params)


# speedup vs baseline: 34.1626x; 1.0351x over previous
import functools

import jax
import jax.numpy as jnp
from jax import lax
from jax.experimental import pallas as pl
from jax.experimental.pallas import tpu as pltpu

IN_DIM = 10
HID_DIM = 10
OUT_DIM = 5
LANES = 128

_W1_OFF = 0
_B1_OFF = 16
_W2_OFF = 24
_B2_OFF = 152

_CONTRACT0 = (((0,), (0,)), ((), ()))


def _mlp_t_kernel(x_ref, p_ref, o_ref):
    x = x_ref[...]
    w1 = p_ref[_W1_OFF:_W1_OFF + IN_DIM, :HID_DIM]
    b1 = p_ref[_B1_OFF:_B1_OFF + 1, :HID_DIM]
    w2 = p_ref[_W2_OFF:_W2_OFF + HID_DIM, :OUT_DIM]
    b2 = p_ref[_B2_OFF:_B2_OFF + 1, :OUT_DIM]

    ones = jnp.ones((1, x.shape[1]), jnp.float32)
    f32 = jnp.float32
    h = (lax.dot_general(w1, x, _CONTRACT0, preferred_element_type=f32)
         + lax.dot_general(b1, ones, _CONTRACT0, preferred_element_type=f32))
    h = jnp.maximum(h, 0.0)
    y = (lax.dot_general(w2, h, _CONTRACT0, preferred_element_type=f32)
         + lax.dot_general(b2, ones, _CONTRACT0, preferred_element_type=f32))
    o_ref[...] = y.astype(o_ref.dtype)


@functools.partial(jax.jit, static_argnames=("tile_b",))
def _forward(x, packed_params, *, tile_b=262144):
    B = x.shape[0]
    xt = x.T
    prows = packed_params.shape[0]

    tile = min(tile_b, ((B + LANES - 1) // LANES) * LANES)
    grid = (pl.cdiv(B, tile),)

    cost = pl.CostEstimate(
        flops=2 * B * (IN_DIM + 1) * (HID_DIM + OUT_DIM),
        transcendentals=0,
        bytes_accessed=B * (IN_DIM + OUT_DIM) * 4 + prows * LANES * 4,
    )

    yt = pl.pallas_call(
        _mlp_t_kernel,
        out_shape=jax.ShapeDtypeStruct((OUT_DIM, B), jnp.float32),
        grid=grid,
        in_specs=[
            pl.BlockSpec((IN_DIM, tile), lambda i: (0, i)),
            pl.BlockSpec((prows, LANES), lambda i: (0, 0)),
        ],
        out_specs=pl.BlockSpec((OUT_DIM, tile), lambda i: (0, i)),
        compiler_params=pltpu.CompilerParams(
            dimension_semantics=("parallel",),
        ),
        cost_estimate=cost,
    )(xt, packed_params)

    return yt.T


def kernel(x, packed_params):
    return _forward(x, packed_params)


# speedup vs baseline: 34.2878x; 1.0037x over previous
import functools

import jax
import jax.numpy as jnp
from jax import lax
from jax.experimental import pallas as pl
from jax.experimental.pallas import tpu as pltpu

IN_DIM = 10
HID_DIM = 10
OUT_DIM = 5
LANES = 128

_W1_OFF = 0
_B1_OFF = 16
_W2_OFF = 24
_B2_OFF = 152

_CONTRACT0 = (((0,), (0,)), ((), ()))


_CHUNK = 32768


def _mlp_t_kernel(x_ref, p_ref, o_ref):
    w1 = p_ref[_W1_OFF:_W1_OFF + IN_DIM, :HID_DIM]
    b1 = p_ref[_B1_OFF:_B1_OFF + 1, :HID_DIM]
    w2 = p_ref[_W2_OFF:_W2_OFF + HID_DIM, :OUT_DIM]
    b2 = p_ref[_B2_OFF:_B2_OFF + 1, :OUT_DIM]

    tile = x_ref.shape[1]
    chunk = min(_CHUNK, tile)
    f32 = jnp.float32
    ones = jnp.ones((1, chunk), f32)
    for c in range(0, tile, chunk):
        x = x_ref[:, c:c + chunk]
        h = (lax.dot_general(w1, x, _CONTRACT0, preferred_element_type=f32)
             + lax.dot_general(b1, ones, _CONTRACT0, preferred_element_type=f32))
        h = jnp.maximum(h, 0.0)
        y = (lax.dot_general(w2, h, _CONTRACT0, preferred_element_type=f32)
             + lax.dot_general(b2, ones, _CONTRACT0, preferred_element_type=f32))
        o_ref[:, c:c + chunk] = y.astype(o_ref.dtype)


@functools.partial(jax.jit, static_argnames=("tile_b",))
def _forward(x, packed_params, *, tile_b=262144):
    B = x.shape[0]
    xt = x.T
    prows = packed_params.shape[0]

    tile = min(tile_b, ((B + LANES - 1) // LANES) * LANES)
    grid = (pl.cdiv(B, tile),)

    cost = pl.CostEstimate(
        flops=2 * B * (IN_DIM + 1) * (HID_DIM + OUT_DIM),
        transcendentals=0,
        bytes_accessed=B * (IN_DIM + OUT_DIM) * 4 + prows * LANES * 4,
    )

    yt = pl.pallas_call(
        _mlp_t_kernel,
        out_shape=jax.ShapeDtypeStruct((OUT_DIM, B), jnp.float32),
        grid=grid,
        in_specs=[
            pl.BlockSpec((IN_DIM, tile), lambda i: (0, i)),
            pl.BlockSpec((prows, LANES), lambda i: (0, 0)),
        ],
        out_specs=pl.BlockSpec((OUT_DIM, tile), lambda i: (0, i)),
        compiler_params=pltpu.CompilerParams(
            dimension_semantics=("parallel",),
        ),
        cost_estimate=cost,
    )(xt, packed_params)

    return yt.T


def kernel(x, packed_params):
    return _forward(x, packed_params)
